# Initial kernel scaffold; baseline (speedup 1.0000x reference)
import jax
import jax.numpy as jnp
from jax.experimental import pallas as pl


def kernel(x):
    raise NotImplementedError("write your pallas kernel here")


# speedup vs baseline: 4.9614x; 4.9614x over previous
import jax
import jax.numpy as jnp
from jax import lax
from jax.experimental import pallas as pl

_TOPK = 64
_BLK_ROWS = 8
_CHUNKS = 256
_LANES = 128
_I32_MIN = -(2**31)
_I32_MAX = 2**31 - 1


def _tc_body(x_ref, o_ref):
    x = x_ref[...]
    mu = lax.bitcast_convert_type(x, jnp.int32)
    m = mu ^ (lax.shift_right_arithmetic(mu, 31) & jnp.int32(0x7FFFFFFF))

    lo0 = jnp.full((_BLK_ROWS, 1, 1), _I32_MIN, jnp.int32)
    hi0 = jnp.full((_BLK_ROWS, 1, 1), _I32_MAX, jnp.int32)

    def it(_, carry):
        lo, hi = carry
        mid = lo + lax.shift_right_logical(hi - lo, 1)
        cg = jnp.sum((m >= mid).astype(jnp.int32), axis=(1, 2), keepdims=True)
        ge = cg >= _TOPK
        return jnp.where(ge, mid, lo), jnp.where(ge, hi, mid)

    lo, hi = lax.fori_loop(0, 32, it, (lo0, hi0))
    t = lo

    gt = m > t
    eq = m == t
    cgt = jnp.sum(gt.astype(jnp.int32), axis=(1, 2), keepdims=True)
    need = (_TOPK - cgt).astype(jnp.float32)

    eqf = eq.astype(jnp.float32)
    ii = lax.broadcasted_iota(jnp.int32, (_LANES, _LANES), 0)
    jj = lax.broadcasted_iota(jnp.int32, (_LANES, _LANES), 1)
    u_lane = (ii < jj).astype(jnp.float32)
    within = lax.dot_general(eqf, u_lane, (((2,), (0,)), ((), ())),
                             preferred_element_type=jnp.float32)
    tot = jnp.sum(eqf, axis=2)
    ci = lax.broadcasted_iota(jnp.int32, (_CHUNKS, _CHUNKS), 0)
    cj = lax.broadcasted_iota(jnp.int32, (_CHUNKS, _CHUNKS), 1)
    u_chunk = (ci < cj).astype(jnp.float32)
    cpre = lax.dot_general(tot, u_chunk, (((1,), (0,)), ((), ())),
                           preferred_element_type=jnp.float32)
    prefix = within + cpre[:, :, None]

    keep = jnp.logical_or(gt, jnp.logical_and(eq, prefix < need))
    o_ref[...] = jnp.where(keep, jnp.maximum(x, 0.0), 0.0)


def kernel(x):
    rows, cols = x.shape
    x4 = x.reshape(rows, _CHUNKS, _LANES)
    grid = rows // _BLK_ROWS
    out = pl.pallas_call(
        _tc_body,
        grid=(grid,),
        in_specs=[pl.BlockSpec((_BLK_ROWS, _CHUNKS, _LANES),
                               lambda i: (i, 0, 0))],
        out_specs=pl.BlockSpec((_BLK_ROWS, _CHUNKS, _LANES),
                               lambda i: (i, 0, 0)),
        out_shape=jax.ShapeDtypeStruct((rows, _CHUNKS, _LANES), x.dtype),
    )(x4)
    return out.reshape(rows, cols)
